# Initial kernel scaffold; baseline (speedup 1.0000x reference)
import jax
import jax.numpy as jnp
from jax.experimental import pallas as pl


def kernel(rec_node_cat, rec_node_scal, rec_edge_cat, rec_edge_scal, rec_edge_src, rec_edge_dst, rec_graph_ids, lig_node_cat, lig_node_scal, lig_edge_cat, lig_edge_scal, lig_edge_src, lig_edge_dst, lig_graph_ids, params):
    raise NotImplementedError("write your pallas kernel here")


# speedup vs baseline: 1.0094x; 1.0094x over previous
import functools

import jax
import jax.numpy as jnp
from jax.experimental import pallas as pl
from jax.experimental.pallas import tpu as pltpu

REC_N = 10000; REC_E = 40000; LIG_N = 640; LIG_E = 2560; B = 16; D = 32


def _cat_scal_embed(tables, cat, scal):
    embs = [tables[c][cat[:, c]] for c in range(tables.shape[0])]
    return jnp.concatenate(embs + [scal], axis=-1)


def _gru(x, h, Wi, bi, Wh, bh):
    gi = x @ Wi + bi
    gh = h @ Wh + bh
    ir, iz, i_n = jnp.split(gi, 3, axis=-1)
    hr, hz, hn = jnp.split(gh, 3, axis=-1)
    r = jax.nn.sigmoid(ir + hr)
    z = jax.nn.sigmoid(iz + hz)
    n = jnp.tanh(i_n + r * hn)
    return (1.0 - z) * n + z * h


def _mpnn(p, pre, h_in, efeat, src, dst, n_nodes, n_layers):
    h = jax.nn.relu(h_in @ p[pre + '_proj_W'] + p[pre + '_proj_b'])
    hidden = h
    ew = jax.nn.relu(efeat @ p[pre + '_enW1'] + p[pre + '_enb1']) @ p[pre + '_enW2'] + p[pre + '_enb2']
    ew = ew.reshape(-1, D, D)
    for _ in range(n_layers):
        msg = jnp.einsum('ei,eio->eo', h[src], ew)
        agg = jax.ops.segment_sum(msg, dst, num_segments=n_nodes) + p[pre + '_nn_b']
        h2 = jax.nn.relu(agg)
        hidden = _gru(h2, hidden, p[pre + '_Wi'], p[pre + '_bi'], p[pre + '_Wh'], p[pre + '_bh'])
        h = hidden
    return h


def _readout(h, gids, W, b):
    w = jax.nn.sigmoid(h @ W + b)
    ws = jax.ops.segment_sum(w * h, gids, num_segments=B)
    mx = jax.ops.segment_max(h, gids, num_segments=B)
    return jnp.concatenate([ws, mx], axis=-1)


def _head_kernel(rr_ref, lr_ref, w1_ref, b1_ref, g1_ref, bb1_ref,
                 w2_ref, b2_ref, g2_ref, bb2_ref, ow_ref, ob_ref, out_ref):
    rr = rr_ref[...]
    lr = lr_ref[...]
    C = jnp.dot(lr, w1_ref[...], preferred_element_type=jnp.float32)
    y = jnp.zeros((B, 256), jnp.float32)
    for i in range(64):
        y = y + rr[:, i:i + 1] * C[:, i * 256:(i + 1) * 256]
    y = y + b1_ref[...]
    mu = jnp.mean(y, axis=-1, keepdims=True)
    v = jnp.mean((y - mu) * (y - mu), axis=-1, keepdims=True)
    y = (y - mu) * jax.lax.rsqrt(v + 1e-5) * g1_ref[...] + bb1_ref[...]
    y = jnp.where(y > 0, y, 0.01 * y)
    y = jnp.dot(y, w2_ref[...], preferred_element_type=jnp.float32) + b2_ref[...]
    mu = jnp.mean(y, axis=-1, keepdims=True)
    v = jnp.mean((y - mu) * (y - mu), axis=-1, keepdims=True)
    y = (y - mu) * jax.lax.rsqrt(v + 1e-5) * g2_ref[...] + bb2_ref[...]
    y = jnp.where(y > 0, y, 0.01 * y)
    out_ref[...] = jnp.dot(y, ow_ref[...], preferred_element_type=jnp.float32) + ob_ref[...]


def _head(rr, lr, p):
    w1v = p['mlp_W1'].reshape(64, 64, 256).transpose(1, 0, 2).reshape(64, 64 * 256)
    out = pl.pallas_call(
        _head_kernel,
        out_shape=jax.ShapeDtypeStruct((B, 1), jnp.float32),
    )(rr, lr, w1v, p['mlp_b1'].reshape(1, 256), p['ln1_g'].reshape(1, 256),
      p['ln1_b'].reshape(1, 256), p['mlp_W2'], p['mlp_b2'].reshape(1, 64),
      p['ln2_g'].reshape(1, 64), p['ln2_b'].reshape(1, 64), p['out_W'],
      p['out_b'].reshape(1, 1))
    return out[:, 0]


@jax.jit
def kernel(rec_node_cat, rec_node_scal, rec_edge_cat, rec_edge_scal, rec_edge_src, rec_edge_dst, rec_graph_ids, lig_node_cat, lig_node_scal, lig_edge_cat, lig_edge_scal, lig_edge_src, lig_edge_dst, lig_graph_ids, params):
    p = params
    rec_hid = _cat_scal_embed(p['rec_node_emb'], rec_node_cat, rec_node_scal)
    lig_hid = _cat_scal_embed(p['lig_node_emb'], lig_node_cat, lig_node_scal)
    rec_ef = _cat_scal_embed(p['rec_edge_emb'], rec_edge_cat, rec_edge_scal)
    lig_ef = _cat_scal_embed(p['lig_edge_emb'], lig_edge_cat, lig_edge_scal)
    rec_out = _mpnn(p, 'rec', rec_hid, rec_ef, rec_edge_src, rec_edge_dst, REC_N, 2)
    lig_out = _mpnn(p, 'lig', lig_hid, lig_ef, lig_edge_src, lig_edge_dst, LIG_N, 3)
    rr = _readout(rec_out, rec_graph_ids, p['rec_rw_W'], p['rec_rw_b'])
    lr = _readout(lig_out, lig_graph_ids, p['lig_rw_W'], p['lig_rw_b'])
    return _head(rr, lr, p)
